# Initial kernel scaffold; baseline (speedup 1.0000x reference)
import jax
import jax.numpy as jnp
from jax.experimental import pallas as pl


def kernel(cate_0, cate_1, cate_2, cate_3, cate_4, cate_5, cate_6, cate_7, cate_8, cate_9, cate_10, cate_11, cate_12, cate_13, cate_14, cate_15, cate_16, cate_17, cate_18, cate_19, cate_20, cate_21, cate_22, cate_23, cate_24, cate_25, emb_0, emb_1, emb_2, emb_3, emb_4, emb_5, emb_6, emb_7, emb_8, emb_9, emb_10, emb_11, emb_12, emb_13, emb_14, emb_15, emb_16, emb_17, emb_18, emb_19, emb_20, emb_21, emb_22, emb_23, emb_24, emb_25, dense_feats, W, b):
    raise NotImplementedError("write your pallas kernel here")


# speedup vs baseline: 1.0658x; 1.0658x over previous
import functools

import jax
import jax.numpy as jnp
from jax import lax
from jax.experimental import pallas as pl
from jax.experimental.pallas import tpu as pltpu
from jax.experimental.pallas import tpu_sc as plsc

B = 16384
V = 1000000
NF = 26
ND = 13
L = 16
NW = 32
BW = B // NW
CH = 128
NC_CHUNK = BW // CH
NG = BW // L


def _body(idx_hbm, dns_hbm, w_hbm, b_hbm, *rest):
    embs = rest[:NF]
    out_hbm = rest[NF]
    idx_v, dns_v, w_v, b_v, gbuf, acc_v, sem = rest[NF + 1:]

    wid = lax.axis_index("s") * 2 + lax.axis_index("c")
    base = wid * BW

    pltpu.sync_copy(idx_hbm.at[wid], idx_v)
    pltpu.sync_copy(dns_hbm.at[wid], dns_v)
    pltpu.sync_copy(w_hbm, w_v)
    pltpu.sync_copy(b_hbm, b_v)

    waits = []
    for i in range(NF):
        for c in range(NC_CHUNK):
            waits.append(
                pltpu.async_copy(embs[i].at[idx_v.at[i, c]], gbuf.at[i, c], sem))
    for w in waits:
        w.wait()

    for g in range(NG):
        c, r = g // (CH // L), (g % (CH // L)) * L
        v = b_v[...]
        for d in range(ND):
            v = v + dns_v[d, pl.ds(g * L, L)] * w_v[d]
        for i in range(NF):
            v = v + gbuf[i, c, pl.ds(r, L)]
        acc_v[pl.ds(g * L, L)] = v

    pltpu.sync_copy(acc_v, out_hbm.at[pl.ds(base, BW)])


@jax.jit
def _run(idx_r, dns_r, w16, b16, *embs):
    mesh = plsc.VectorSubcoreMesh(core_axis_name="c", subcore_axis_name="s")
    kfn = pl.kernel(
        _body,
        out_type=jax.ShapeDtypeStruct((B,), jnp.float32),
        mesh=mesh,
        scratch_types=[
            pltpu.VMEM((NF, NC_CHUNK, CH), jnp.int32),
            pltpu.VMEM((ND, BW), jnp.float32),
            pltpu.VMEM((ND, L), jnp.float32),
            pltpu.VMEM((L,), jnp.float32),
            pltpu.VMEM((NF, NC_CHUNK, CH), jnp.float32),
            pltpu.VMEM((BW,), jnp.float32),
            pltpu.SemaphoreType.DMA,
        ],
    )
    return kfn(idx_r, dns_r, w16, b16, *embs)


def kernel(*args):
    cates = args[:NF]
    embs = args[NF:2 * NF]
    dense_feats, W, b = args[2 * NF:]

    idx = jnp.stack([c.reshape(B) for c in cates])
    idx_r = idx.reshape(NF, NW, NC_CHUNK, CH).transpose(1, 0, 2, 3)
    dns_r = dense_feats.T.reshape(ND, NW, BW).transpose(1, 0, 2)
    w16 = jnp.broadcast_to(W.reshape(ND, 1), (ND, L))
    b16 = jnp.broadcast_to(b.reshape(1), (L,))
    flat_embs = [e.reshape(V) for e in embs]

    out = _run(idx_r, dns_r, w16, b16, *flat_embs)
    return out.reshape(B, 1)
